# Initial kernel scaffold; baseline (speedup 1.0000x reference)
import jax
import jax.numpy as jnp
from jax.experimental import pallas as pl


def kernel(x, edge_index, W_l, b_l, W_r, ln_gamma, ln_beta):
    raise NotImplementedError("write your pallas kernel here")


# speedup vs baseline: 5.5447x; 5.5447x over previous
import dataclasses
import functools
import math

import jax
import jax.numpy as jnp
from jax import lax
from jax.experimental import pallas as pl
from jax.experimental.pallas import tpu as pltpu
from jax.experimental.pallas import tpu_sc as plsc

NC = 2
NS = 16
NW = NC * NS


def _sc_aggregate(src, dst, x, zeros2d, zeros1d):
    n, d = x.shape
    e = src.shape[0]
    epw = e // NW
    ch = 80
    nchunk = epw // ch
    rps = (n // NS) // 8 * 8
    tail = n - NS * rps

    mesh = plsc.VectorSubcoreMesh(
        core_axis_name="c", subcore_axis_name="s", num_cores=NC,
        num_subcores=NS)

    cp = pltpu.CompilerParams()
    if "needs_layout_passes" in pltpu.CompilerParams.__dataclass_fields__:
        cp = dataclasses.replace(cp, needs_layout_passes=False)

    @functools.partial(
        pl.kernel,
        out_type=(
            jax.ShapeDtypeStruct((NC, n, d), jnp.float32),
            jax.ShapeDtypeStruct((NW, n), jnp.float32),
        ),
        mesh=mesh,
        scratch_types=[
            pltpu.VMEM((ch,), jnp.int32),
            pltpu.VMEM((1, ch), jnp.int32),
            pltpu.VMEM((ch,), jnp.int32),
            pltpu.VMEM((ch, d), jnp.float32),
            pltpu.VMEM((n,), jnp.float32),
            pltpu.VMEM_SHARED((n, d), jnp.float32),
            pltpu.SemaphoreType.DMA,
        ],
        compiler_params=cp,
    )
    def sc_agg(src_hbm, dst_hbm, x_hbm, z2_hbm, z1_hbm, part_hbm, cnt_hbm,
               src_v, dst2_v, dst1_v, rows_v, cnt_v, acc_sh, sem):
        c = lax.axis_index("c")
        s = lax.axis_index("s")
        wid = c * NS + s
        pltpu.sync_copy(z1_hbm, cnt_v)
        pltpu.sync_copy(z2_hbm.at[pl.ds(s * rps, rps)],
                        acc_sh.at[pl.ds(s * rps, rps)])

        @pl.when(s == 0)
        def _():
            pltpu.sync_copy(z2_hbm.at[pl.ds(NS * rps, tail)],
                            acc_sh.at[pl.ds(NS * rps, tail)])

        plsc.subcore_barrier()

        ones = jnp.ones((16,), jnp.float32)

        @pl.loop(0, nchunk)
        def _(j):
            off = wid * epw + j * ch
            pltpu.sync_copy(src_hbm.at[pl.ds(off, ch)], src_v)
            pltpu.sync_copy(dst_hbm.at[pl.ds(off, ch)], dst2_v.at[0])
            pltpu.sync_copy(dst_hbm.at[pl.ds(off, ch)], dst1_v)
            pltpu.async_copy(x_hbm.at[src_v], rows_v, sem).wait()
            pltpu.sync_copy(rows_v, acc_sh.at[dst2_v.at[0]], add=True)
            for k in range(ch // 16):
                dk = dst1_v[pl.ds(k * 16, 16)]
                plsc.addupdate_scatter(cnt_v, [dk], ones)

        plsc.subcore_barrier()
        pltpu.sync_copy(acc_sh.at[pl.ds(s * rps, rps)],
                        part_hbm.at[c].at[pl.ds(s * rps, rps)])

        @pl.when(s == 0)
        def _():
            pltpu.sync_copy(acc_sh.at[pl.ds(NS * rps, tail)],
                            part_hbm.at[c].at[pl.ds(NS * rps, tail)])

        pltpu.sync_copy(cnt_v, cnt_hbm.at[wid])

    return sc_agg(src, dst, x, zeros2d, zeros1d)


def _tc_combine(part_ref, cntp_ref, x_ref, wl_ref, bl_ref, wr_ref, g_ref,
                b_ref, o_ref):
    agg = part_ref[0] + part_ref[1]
    cnt = jnp.sum(cntp_ref[...], axis=1)
    mean = agg / jnp.maximum(cnt, 1.0)[:, None]
    h = (jnp.dot(mean, wl_ref[...], preferred_element_type=jnp.float32)
         + jnp.dot(x_ref[...], wr_ref[...], preferred_element_type=jnp.float32)
         + bl_ref[...])
    mu = jnp.mean(h, axis=1, keepdims=True)
    hc = h - mu
    var = jnp.mean(hc * hc, axis=1, keepdims=True)
    hn = hc * lax.rsqrt(var + 1e-5) * g_ref[...] + b_ref[...]
    o_ref[...] = 0.5 * hn * (1.0 + lax.erf(hn * (1.0 / math.sqrt(2.0))))


def kernel(x, edge_index, W_l, b_l, W_r, ln_gamma, ln_beta):
    n, d = x.shape
    src = edge_index[0]
    dst = edge_index[1]
    zeros2d = jnp.zeros((n, d), jnp.float32)
    zeros1d = jnp.zeros((n,), jnp.float32)
    part, cntp = _sc_aggregate(src, dst, x, zeros2d, zeros1d)

    blk = 1000
    grid = (n // blk,)
    out = pl.pallas_call(
        _tc_combine,
        grid=grid,
        in_specs=[
            pl.BlockSpec((NC, blk, d), lambda i: (0, i, 0)),
            pl.BlockSpec((blk, NW), lambda i: (i, 0)),
            pl.BlockSpec((blk, d), lambda i: (i, 0)),
            pl.BlockSpec((d, d), lambda i: (0, 0)),
            pl.BlockSpec((1, d), lambda i: (0, 0)),
            pl.BlockSpec((d, d), lambda i: (0, 0)),
            pl.BlockSpec((1, d), lambda i: (0, 0)),
            pl.BlockSpec((1, d), lambda i: (0, 0)),
        ],
        out_specs=pl.BlockSpec((blk, d), lambda i: (i, 0)),
        out_shape=jax.ShapeDtypeStruct((n, d), jnp.float32),
    )(part, cntp.T, x, W_l, b_l.reshape(1, d), W_r, ln_gamma.reshape(1, d),
      ln_beta.reshape(1, d))
    return out


# speedup vs baseline: 11.6656x; 2.1039x over previous
import dataclasses
import functools
import math

import jax
import jax.numpy as jnp
from jax import lax
from jax.experimental import pallas as pl
from jax.experimental.pallas import tpu as pltpu
from jax.experimental.pallas import tpu_sc as plsc

NC = 2
NS = 16
NW = NC * NS


def _sc_aggregate(src, dst, x, zeros2d, zeros1d):
    n, d = x.shape
    e = src.shape[0]
    epw = e // NW
    ch = 96
    nfull = epw // ch
    tail_e = epw - nfull * ch
    rps = (n // NS) // 8 * 8
    tail = n - NS * rps

    mesh = plsc.VectorSubcoreMesh(
        core_axis_name="c", subcore_axis_name="s", num_cores=NC,
        num_subcores=NS)

    cp = pltpu.CompilerParams()
    if "needs_layout_passes" in pltpu.CompilerParams.__dataclass_fields__:
        cp = dataclasses.replace(cp, needs_layout_passes=False)

    @functools.partial(
        pl.kernel,
        out_type=(
            jax.ShapeDtypeStruct((NC, n, d), jnp.float32),
            jax.ShapeDtypeStruct((NW, n), jnp.float32),
        ),
        mesh=mesh,
        scratch_types=[
            pltpu.VMEM((epw,), jnp.int32),
            pltpu.VMEM((2, ch), jnp.int32),
            pltpu.VMEM((2, ch), jnp.int32),
            pltpu.VMEM((1, 16), jnp.int32),
            pltpu.VMEM((2, ch, d), jnp.float32),
            pltpu.VMEM((n,), jnp.float32),
            pltpu.VMEM_SHARED((n, d), jnp.float32),
            pltpu.SemaphoreType.DMA,
            pltpu.SemaphoreType.DMA,
            pltpu.SemaphoreType.DMA,
            pltpu.SemaphoreType.DMA,
        ],
        compiler_params=cp,
    )
    def sc_agg(src_hbm, dst_hbm, x_hbm, z2_hbm, z1_hbm, part_hbm, cnt_hbm,
               dsta_v, sring_v, dring_v, dtail_v, rows_v, cnt_v, acc_sh,
               sem_g0, sem_g1, sem_i0, sem_i1):
        c = lax.axis_index("c")
        s = lax.axis_index("s")
        wid = c * NS + s
        base = wid * epw
        pltpu.sync_copy(z1_hbm, cnt_v)
        pltpu.sync_copy(z2_hbm.at[pl.ds(s * rps, rps)],
                        acc_sh.at[pl.ds(s * rps, rps)])

        @pl.when(s == 0)
        def _():
            pltpu.sync_copy(z2_hbm.at[pl.ds(NS * rps, tail)],
                            acc_sh.at[pl.ds(NS * rps, tail)])

        pltpu.sync_copy(dst_hbm.at[pl.ds(base, epw)], dsta_v)
        plsc.subcore_barrier()

        ones = jnp.ones((16,), jnp.float32)
        gsems = (sem_g0, sem_g1)
        isems = (sem_i0, sem_i1)

        def stage(j, b, width):
            ring = dring_v if width == ch else dtail_v
            for k in range(width // 16):
                dk = dsta_v[pl.ds(j * ch + k * 16, 16)]
                ring[b, pl.ds(k * 16, 16)] = dk
                plsc.addupdate_scatter(cnt_v, [dk], ones)

        def idx_copy(j, b):
            return pltpu.make_async_copy(
                src_hbm.at[pl.ds(base + j * ch, ch)], sring_v.at[b], isems[b])

        def gather_copy(b):
            return pltpu.make_async_copy(
                x_hbm.at[sring_v.at[b]], rows_v.at[b], gsems[b])

        def body(j, b, last, prefetch=True):
            stage(j, b, ch)
            gather_copy(b).wait()
            if not last:
                idx_copy(j + 1, b ^ 1).wait()
                gather_copy(b ^ 1).start()
            if prefetch and not last:
                idx_copy(j + 2, b).start()
            pltpu.sync_copy(rows_v.at[b], acc_sh.at[dring_v.at[b]], add=True)

        npair = nfull // 2
        ic0 = idx_copy(0, 0)
        ic0.start()
        ic0.wait()
        gather_copy(0).start()
        idx_copy(1, 1).start()

        @pl.loop(0, npair - 1)
        def _(p):
            for b in range(2):
                body(2 * p + b, b, False)

        body(nfull - 2, 0, False, prefetch=False)
        body(nfull - 1, 1, True)

        if tail_e:
            pltpu.sync_copy(src_hbm.at[pl.ds(base + nfull * ch, tail_e)],
                            sring_v.at[0].at[pl.ds(0, tail_e)])
            stage(nfull, 0, tail_e)
            pltpu.async_copy(
                x_hbm.at[sring_v.at[0].at[pl.ds(0, tail_e)]],
                rows_v.at[0].at[pl.ds(0, tail_e)], sem_g0).wait()
            pltpu.sync_copy(rows_v.at[0].at[pl.ds(0, tail_e)],
                            acc_sh.at[dtail_v.at[0]], add=True)

        plsc.subcore_barrier()
        pltpu.sync_copy(acc_sh.at[pl.ds(s * rps, rps)],
                        part_hbm.at[c].at[pl.ds(s * rps, rps)])

        @pl.when(s == 0)
        def _():
            pltpu.sync_copy(acc_sh.at[pl.ds(NS * rps, tail)],
                            part_hbm.at[c].at[pl.ds(NS * rps, tail)])

        pltpu.sync_copy(cnt_v, cnt_hbm.at[wid])

    return sc_agg(src, dst, x, zeros2d, zeros1d)


def _tc_combine(part_ref, cntp_ref, x_ref, wl_ref, bl_ref, wr_ref, g_ref,
                b_ref, o_ref):
    agg = part_ref[0] + part_ref[1]
    cnt = jnp.sum(cntp_ref[...], axis=1)
    mean = agg / jnp.maximum(cnt, 1.0)[:, None]
    h = (jnp.dot(mean, wl_ref[...], preferred_element_type=jnp.float32)
         + jnp.dot(x_ref[...], wr_ref[...], preferred_element_type=jnp.float32)
         + bl_ref[...])
    mu = jnp.mean(h, axis=1, keepdims=True)
    hc = h - mu
    var = jnp.mean(hc * hc, axis=1, keepdims=True)
    hn = hc * lax.rsqrt(var + 1e-5) * g_ref[...] + b_ref[...]
    o_ref[...] = 0.5 * hn * (1.0 + lax.erf(hn * (1.0 / math.sqrt(2.0))))


def kernel(x, edge_index, W_l, b_l, W_r, ln_gamma, ln_beta):
    n, d = x.shape
    src = edge_index[0]
    dst = edge_index[1]
    zeros2d = jnp.zeros((n, d), jnp.float32)
    zeros1d = jnp.zeros((n,), jnp.float32)
    part, cntp = _sc_aggregate(src, dst, x, zeros2d, zeros1d)

    blk = 1000
    grid = (n // blk,)
    out = pl.pallas_call(
        _tc_combine,
        grid=grid,
        in_specs=[
            pl.BlockSpec((NC, blk, d), lambda i: (0, i, 0)),
            pl.BlockSpec((blk, NW), lambda i: (i, 0)),
            pl.BlockSpec((blk, d), lambda i: (i, 0)),
            pl.BlockSpec((d, d), lambda i: (0, 0)),
            pl.BlockSpec((1, d), lambda i: (0, 0)),
            pl.BlockSpec((d, d), lambda i: (0, 0)),
            pl.BlockSpec((1, d), lambda i: (0, 0)),
            pl.BlockSpec((1, d), lambda i: (0, 0)),
        ],
        out_specs=pl.BlockSpec((blk, d), lambda i: (i, 0)),
        out_shape=jax.ShapeDtypeStruct((n, d), jnp.float32),
    )(part, cntp.T, x, W_l, b_l.reshape(1, d), W_r, ln_gamma.reshape(1, d),
      ln_beta.reshape(1, d))
    return out


# speedup vs baseline: 12.5113x; 1.0725x over previous
import dataclasses
import functools
import math

import jax
import jax.numpy as jnp
from jax import lax
from jax.experimental import pallas as pl
from jax.experimental.pallas import tpu as pltpu
from jax.experimental.pallas import tpu_sc as plsc

NC = 2
NS = 16
NW = NC * NS


def _sc_aggregate(src, dst, x, zeros2d, zeros1d):
    n, d = x.shape
    e = src.shape[0]
    epw = e // NW
    ch = 96
    nfull = epw // ch
    tail_e = epw - nfull * ch
    R = 3
    rps = (n // NS) // 8 * 8
    tail = n - NS * rps

    mesh = plsc.VectorSubcoreMesh(
        core_axis_name="c", subcore_axis_name="s", num_cores=NC,
        num_subcores=NS)

    cp = pltpu.CompilerParams()
    if "needs_layout_passes" in pltpu.CompilerParams.__dataclass_fields__:
        cp = dataclasses.replace(cp, needs_layout_passes=False)

    @functools.partial(
        pl.kernel,
        out_type=(
            jax.ShapeDtypeStruct((NC, n, d), jnp.float32),
            jax.ShapeDtypeStruct((NW, n), jnp.float32),
        ),
        mesh=mesh,
        scratch_types=[
            pltpu.VMEM((R, ch), jnp.int32),
            pltpu.VMEM((R, ch), jnp.int32),
            pltpu.VMEM((1, 16), jnp.int32),
            pltpu.VMEM((R, ch, d), jnp.float32),
            pltpu.VMEM((n,), jnp.float32),
            pltpu.VMEM_SHARED((n, d), jnp.float32),
            pltpu.SemaphoreType.DMA,
            pltpu.SemaphoreType.DMA,
            pltpu.SemaphoreType.DMA,
            pltpu.SemaphoreType.DMA,
            pltpu.SemaphoreType.DMA,
            pltpu.SemaphoreType.DMA,
        ],
        compiler_params=cp,
    )
    def sc_agg(src_hbm, dst_hbm, x_hbm, z2_hbm, z1_hbm, part_hbm, cnt_hbm,
               sring_v, dring_v, dtail_v, rows_v, cnt_v, acc_sh,
               sem_g0, sem_g1, sem_g2, sem_i0, sem_i1, sem_i2):
        c = lax.axis_index("c")
        s = lax.axis_index("s")
        wid = c * NS + s
        base = wid * epw
        pltpu.sync_copy(z1_hbm, cnt_v)
        pltpu.sync_copy(z2_hbm.at[pl.ds(s * rps, rps)],
                        acc_sh.at[pl.ds(s * rps, rps)])

        @pl.when(s == 0)
        def _():
            pltpu.sync_copy(z2_hbm.at[pl.ds(NS * rps, tail)],
                            acc_sh.at[pl.ds(NS * rps, tail)])

        plsc.subcore_barrier()

        ones = jnp.ones((16,), jnp.float32)
        gsems = (sem_g0, sem_g1, sem_g2)
        isems = (sem_i0, sem_i1, sem_i2)

        def hist(b, width, ring):
            for k in range(width // 16):
                dk = ring[b, pl.ds(k * 16, 16)]
                plsc.addupdate_scatter(cnt_v, [dk], ones)

        def idx_copy(j, b):
            return (
                pltpu.make_async_copy(src_hbm.at[pl.ds(base + j * ch, ch)],
                                      sring_v.at[b], isems[b]),
                pltpu.make_async_copy(dst_hbm.at[pl.ds(base + j * ch, ch)],
                                      dring_v.at[b], isems[b]),
            )

        def idx_start(j, b):
            for cp_ in idx_copy(j, b):
                cp_.start()

        def idx_wait(j, b):
            for cp_ in idx_copy(j, b):
                cp_.wait()

        def gather_copy(b):
            return pltpu.make_async_copy(
                x_hbm.at[sring_v.at[b]], rows_v.at[b], gsems[b])

        def body(j, b, ahead2=True, pref=True):
            gather_copy(b).wait()
            if ahead2:
                b2 = (b + 2) % R
                idx_wait(j + 2, b2)
                gather_copy(b2).start()
            pltpu.sync_copy(rows_v.at[b], acc_sh.at[dring_v.at[b]], add=True)
            hist(b, ch, dring_v)
            if pref:
                idx_start(j + 3, b)

        for j0 in range(R):
            idx_start(j0, j0)
        for j0 in range(2):
            idx_wait(j0, j0)
            gather_copy(j0).start()

        ntrip = nfull // R
        npeel = nfull - 3 * (ntrip - 1)

        @pl.loop(0, ntrip - 1)
        def _(p):
            for b in range(R):
                body(R * p + b, b)

        for j in range(nfull - npeel, nfull):
            body(j, j % R, ahead2=(j + 2 < nfull), pref=(j + 3 < nfull))

        if tail_e:
            pltpu.sync_copy(src_hbm.at[pl.ds(base + nfull * ch, tail_e)],
                            sring_v.at[0].at[pl.ds(0, tail_e)])
            pltpu.sync_copy(dst_hbm.at[pl.ds(base + nfull * ch, tail_e)],
                            dtail_v.at[0])
            pltpu.async_copy(
                x_hbm.at[sring_v.at[0].at[pl.ds(0, tail_e)]],
                rows_v.at[0].at[pl.ds(0, tail_e)], sem_g0).wait()
            pltpu.sync_copy(rows_v.at[0].at[pl.ds(0, tail_e)],
                            acc_sh.at[dtail_v.at[0]], add=True)
            hist(0, tail_e, dtail_v)

        plsc.subcore_barrier()
        pltpu.sync_copy(acc_sh.at[pl.ds(s * rps, rps)],
                        part_hbm.at[c].at[pl.ds(s * rps, rps)])

        @pl.when(s == 0)
        def _():
            pltpu.sync_copy(acc_sh.at[pl.ds(NS * rps, tail)],
                            part_hbm.at[c].at[pl.ds(NS * rps, tail)])

        pltpu.sync_copy(cnt_v, cnt_hbm.at[wid])

    return sc_agg(src, dst, x, zeros2d, zeros1d)


def _tc_combine(part_ref, cntp_ref, x_ref, wl_ref, bl_ref, wr_ref, g_ref,
                b_ref, o_ref):
    agg = part_ref[0] + part_ref[1]
    cnt = jnp.sum(cntp_ref[...], axis=1)
    mean = agg / jnp.maximum(cnt, 1.0)[:, None]
    h = (jnp.dot(mean, wl_ref[...], preferred_element_type=jnp.float32)
         + jnp.dot(x_ref[...], wr_ref[...], preferred_element_type=jnp.float32)
         + bl_ref[...])
    mu = jnp.mean(h, axis=1, keepdims=True)
    hc = h - mu
    var = jnp.mean(hc * hc, axis=1, keepdims=True)
    hn = hc * lax.rsqrt(var + 1e-5) * g_ref[...] + b_ref[...]
    o_ref[...] = 0.5 * hn * (1.0 + lax.erf(hn * (1.0 / math.sqrt(2.0))))


def kernel(x, edge_index, W_l, b_l, W_r, ln_gamma, ln_beta):
    n, d = x.shape
    src = edge_index[0]
    dst = edge_index[1]
    zeros2d = jnp.zeros((n, d), jnp.float32)
    zeros1d = jnp.zeros((n,), jnp.float32)
    part, cntp = _sc_aggregate(src, dst, x, zeros2d, zeros1d)

    blk = 1000
    grid = (n // blk,)
    out = pl.pallas_call(
        _tc_combine,
        grid=grid,
        in_specs=[
            pl.BlockSpec((NC, blk, d), lambda i: (0, i, 0)),
            pl.BlockSpec((blk, NW), lambda i: (i, 0)),
            pl.BlockSpec((blk, d), lambda i: (i, 0)),
            pl.BlockSpec((d, d), lambda i: (0, 0)),
            pl.BlockSpec((1, d), lambda i: (0, 0)),
            pl.BlockSpec((d, d), lambda i: (0, 0)),
            pl.BlockSpec((1, d), lambda i: (0, 0)),
            pl.BlockSpec((1, d), lambda i: (0, 0)),
        ],
        out_specs=pl.BlockSpec((blk, d), lambda i: (i, 0)),
        out_shape=jax.ShapeDtypeStruct((n, d), jnp.float32),
    )(part, cntp.T, x, W_l, b_l.reshape(1, d), W_r, ln_gamma.reshape(1, d),
      ln_beta.reshape(1, d))
    return out
